# baseline (device time: 21176 ns/iter reference)
import jax
import jax.numpy as jnp
from jax import lax
from jax.experimental import pallas as pl
from jax.experimental.pallas import tpu as pltpu

N_CHUNKS = 16
DIRECT_TAIL = 2
N_FWD = N_CHUNKS - DIRECT_TAIL
N_STAGE = 4
N_OUT = 8


def kernel(x):
    m, n = x.shape
    half = m // 2
    ck = half // N_CHUNKS
    stage_rows = m // N_STAGE
    out_rows = m // N_OUT
    adds_per_stage = stage_rows // ck
    adds_per_out = out_rows // ck

    def body(x_hbm, out_hbm, xv, ov, ybuf, xbuf, xv_sems, ov_sems,
             y_send_sems, y_recv_sems, x_send_sems, x_recv_sems):
        my_x = lax.axis_index("x")
        my_y = lax.axis_index("y")
        my_z = lax.axis_index("z")
        y_partner = (my_x, 1 - my_y, my_z)
        x_partner = (1 - my_x, my_y, my_z)
        base = my_x * half
        obase = (1 - my_x) * half

        stage_in = []
        for s in range(N_STAGE):
            rel = (s % (N_STAGE // 2)) * stage_rows
            start = (base if s < N_STAGE // 2 else obase) + rel
            rows = pl.ds(start, stage_rows)
            cp = pltpu.make_async_copy(
                x_hbm.at[rows, :], xv.at[rows, :], xv_sems.at[s]
            )
            cp.start()
            stage_in.append(cp)

        barrier_sem = pltpu.get_barrier_semaphore()
        for nbr in (y_partner, x_partner):
            pl.semaphore_signal(
                barrier_sem, inc=1, device_id=nbr,
                device_id_type=pl.DeviceIdType.MESH,
            )
        pl.semaphore_wait(barrier_sem, 2)

        y_rdmas = []
        for c in range(N_CHUNKS):
            rdma = pltpu.make_async_remote_copy(
                src_ref=x_hbm.at[pl.ds(base + c * ck, ck), :],
                dst_ref=ybuf.at[pl.ds(c * ck, ck), :],
                send_sem=y_send_sems.at[c],
                recv_sem=y_recv_sems.at[c],
                device_id=y_partner,
                device_id_type=pl.DeviceIdType.MESH,
            )
            rdma.start()
            y_rdmas.append(rdma)
        tail_rdmas = []
        for d in range(DIRECT_TAIL):
            c = N_FWD + d
            rdma = pltpu.make_async_remote_copy(
                src_ref=x_hbm.at[pl.ds(obase + c * ck, ck), :],
                dst_ref=xbuf.at[pl.ds(c * ck, ck), :],
                send_sem=y_send_sems.at[N_CHUNKS + d],
                recv_sem=x_recv_sems.at[c],
                device_id=y_partner,
                device_id_type=pl.DeviceIdType.MESH,
            )
            rdma.start()
            tail_rdmas.append(rdma)

        out_dmas = []
        fwd_rdmas = []
        for c in range(N_CHUNKS):
            y_rdmas[c].wait_recv()
            if c < N_FWD:
                rdma = pltpu.make_async_remote_copy(
                    src_ref=ybuf.at[pl.ds(c * ck, ck), :],
                    dst_ref=xbuf.at[pl.ds(c * ck, ck), :],
                    send_sem=x_send_sems.at[c],
                    recv_sem=x_recv_sems.at[c],
                    device_id=x_partner,
                    device_id_type=pl.DeviceIdType.MESH,
                )
                rdma.start()
                fwd_rdmas.append(rdma)
            if c % adds_per_stage == 0:
                stage_in[c // adds_per_stage].wait()
            rows = pl.ds(base + c * ck, ck)
            ov[rows, :] = xv[rows, :] + ybuf[pl.ds(c * ck, ck), :]
            if c % adds_per_out == adds_per_out - 1:
                g = c // adds_per_out
                grows = pl.ds(base + g * out_rows, out_rows)
                cp = pltpu.make_async_copy(
                    ov.at[grows, :], out_hbm.at[grows, :], ov_sems.at[g]
                )
                cp.start()
                out_dmas.append(cp)

        for c in range(N_CHUNKS):
            if c < N_FWD:
                fwd_rdmas[c].wait_recv()
            else:
                tail_rdmas[c - N_FWD].wait_recv()
            if c % adds_per_stage == 0:
                stage_in[N_STAGE // 2 + c // adds_per_stage].wait()
            rows = pl.ds(obase + c * ck, ck)
            ov[rows, :] = xv[rows, :] + xbuf[pl.ds(c * ck, ck), :]
            if c % adds_per_out == adds_per_out - 1:
                g = c // adds_per_out
                grows = pl.ds(obase + g * out_rows, out_rows)
                cp = pltpu.make_async_copy(
                    ov.at[grows, :], out_hbm.at[grows, :],
                    ov_sems.at[N_OUT // 2 + g],
                )
                cp.start()
                out_dmas.append(cp)

        for cp in out_dmas:
            cp.wait()
        for r in y_rdmas + tail_rdmas + fwd_rdmas:
            r.wait_send()

    return pl.pallas_call(
        body,
        out_shape=jax.ShapeDtypeStruct((m, n), x.dtype),
        in_specs=[pl.BlockSpec(memory_space=pl.ANY)],
        out_specs=pl.BlockSpec(memory_space=pl.ANY),
        scratch_shapes=[
            pltpu.VMEM((m, n), x.dtype),
            pltpu.VMEM((m, n), x.dtype),
            pltpu.VMEM((half, n), x.dtype),
            pltpu.VMEM((half, n), x.dtype),
            pltpu.SemaphoreType.DMA((N_STAGE,)),
            pltpu.SemaphoreType.DMA((N_OUT,)),
            pltpu.SemaphoreType.DMA((N_CHUNKS + DIRECT_TAIL,)),
            pltpu.SemaphoreType.DMA((N_CHUNKS,)),
            pltpu.SemaphoreType.DMA((N_FWD,)),
            pltpu.SemaphoreType.DMA((N_CHUNKS,)),
        ],
        compiler_params=pltpu.CompilerParams(collective_id=0),
    )(x)


# device time: 21148 ns/iter; 1.0013x vs baseline; 1.0013x over previous
import jax
import jax.numpy as jnp
from jax import lax
from jax.experimental import pallas as pl
from jax.experimental.pallas import tpu as pltpu

N_CHUNKS = 16
DIRECT_TAIL = 2
N_FWD = N_CHUNKS - DIRECT_TAIL
N_STAGE = 4
N_OUT = 8


def kernel(x):
    m, n = x.shape
    half = m // 2
    ck = half // N_CHUNKS
    stage_rows = m // N_STAGE
    out_rows = m // N_OUT
    adds_per_stage = stage_rows // ck
    adds_per_out = out_rows // ck

    def body(x_hbm, out_hbm, xv, ov, ybuf, xbuf, xv_sems, ov_sems,
             y_send_sems, y_recv_sems, x_send_sems, x_recv_sems):
        my_x = lax.axis_index("x")
        my_y = lax.axis_index("y")
        my_z = lax.axis_index("z")
        y_partner = (my_x, 1 - my_y, my_z)
        x_partner = (1 - my_x, my_y, my_z)
        base = my_x * half
        obase = (1 - my_x) * half

        stage_in = []
        for s in range(N_STAGE):
            rel = (s % (N_STAGE // 2)) * stage_rows
            start = (base if s < N_STAGE // 2 else obase) + rel
            rows = pl.ds(start, stage_rows)
            cp = pltpu.make_async_copy(
                x_hbm.at[rows, :], xv.at[rows, :], xv_sems.at[s]
            )
            cp.start()
            stage_in.append(cp)

        barrier_sem = pltpu.get_barrier_semaphore()
        for nbr in (y_partner, x_partner):
            pl.semaphore_signal(
                barrier_sem, inc=1, device_id=nbr,
                device_id_type=pl.DeviceIdType.MESH,
            )
        pl.semaphore_wait(barrier_sem, 2)

        y_rdmas = []
        for c in range(N_CHUNKS):
            rdma = pltpu.make_async_remote_copy(
                src_ref=x_hbm.at[pl.ds(base + c * ck, ck), :],
                dst_ref=ybuf.at[pl.ds(c * ck, ck), :],
                send_sem=y_send_sems.at[c],
                recv_sem=y_recv_sems.at[c],
                device_id=y_partner,
                device_id_type=pl.DeviceIdType.MESH,
            )
            rdma.start()
            y_rdmas.append(rdma)
        tail_rdmas = []
        for d in range(DIRECT_TAIL):
            c = N_FWD + d
            rdma = pltpu.make_async_remote_copy(
                src_ref=x_hbm.at[pl.ds(obase + c * ck, ck), :],
                dst_ref=xbuf.at[pl.ds(c * ck, ck), :],
                send_sem=y_send_sems.at[N_CHUNKS + d],
                recv_sem=x_recv_sems.at[c],
                device_id=y_partner,
                device_id_type=pl.DeviceIdType.MESH,
            )
            rdma.start()
            tail_rdmas.append(rdma)

        out_dmas = []
        fwd_rdmas = []

        def fold_other_half(k):
            if k < N_FWD:
                fwd_rdmas[k].wait_recv()
            else:
                tail_rdmas[k - N_FWD].wait_recv()
            if k % adds_per_stage == 0:
                stage_in[N_STAGE // 2 + k // adds_per_stage].wait()
            rows = pl.ds(obase + k * ck, ck)
            ov[rows, :] = xv[rows, :] + xbuf[pl.ds(k * ck, ck), :]
            if k % adds_per_out == adds_per_out - 1:
                g = k // adds_per_out
                grows = pl.ds(obase + g * out_rows, out_rows)
                cp = pltpu.make_async_copy(
                    ov.at[grows, :], out_hbm.at[grows, :],
                    ov_sems.at[N_OUT // 2 + g],
                )
                cp.start()
                out_dmas.append(cp)

        FOLD_LAG = 4
        for c in range(N_CHUNKS):
            y_rdmas[c].wait_recv()
            if c < N_FWD:
                rdma = pltpu.make_async_remote_copy(
                    src_ref=ybuf.at[pl.ds(c * ck, ck), :],
                    dst_ref=xbuf.at[pl.ds(c * ck, ck), :],
                    send_sem=x_send_sems.at[c],
                    recv_sem=x_recv_sems.at[c],
                    device_id=x_partner,
                    device_id_type=pl.DeviceIdType.MESH,
                )
                rdma.start()
                fwd_rdmas.append(rdma)
            if c % adds_per_stage == 0:
                stage_in[c // adds_per_stage].wait()
            rows = pl.ds(base + c * ck, ck)
            ov[rows, :] = xv[rows, :] + ybuf[pl.ds(c * ck, ck), :]
            if c % adds_per_out == adds_per_out - 1:
                g = c // adds_per_out
                grows = pl.ds(base + g * out_rows, out_rows)
                cp = pltpu.make_async_copy(
                    ov.at[grows, :], out_hbm.at[grows, :], ov_sems.at[g]
                )
                cp.start()
                out_dmas.append(cp)
            if c >= FOLD_LAG:
                fold_other_half(c - FOLD_LAG)

        for k in range(N_CHUNKS - FOLD_LAG, N_CHUNKS):
            fold_other_half(k)

        for cp in out_dmas:
            cp.wait()
        for r in y_rdmas + tail_rdmas + fwd_rdmas:
            r.wait_send()

    return pl.pallas_call(
        body,
        out_shape=jax.ShapeDtypeStruct((m, n), x.dtype),
        in_specs=[pl.BlockSpec(memory_space=pl.ANY)],
        out_specs=pl.BlockSpec(memory_space=pl.ANY),
        scratch_shapes=[
            pltpu.VMEM((m, n), x.dtype),
            pltpu.VMEM((m, n), x.dtype),
            pltpu.VMEM((half, n), x.dtype),
            pltpu.VMEM((half, n), x.dtype),
            pltpu.SemaphoreType.DMA((N_STAGE,)),
            pltpu.SemaphoreType.DMA((N_OUT,)),
            pltpu.SemaphoreType.DMA((N_CHUNKS + DIRECT_TAIL,)),
            pltpu.SemaphoreType.DMA((N_CHUNKS,)),
            pltpu.SemaphoreType.DMA((N_FWD,)),
            pltpu.SemaphoreType.DMA((N_CHUNKS,)),
        ],
        compiler_params=pltpu.CompilerParams(collective_id=0),
    )(x)


# device time: 20877 ns/iter; 1.0143x vs baseline; 1.0130x over previous
import jax
import jax.numpy as jnp
from jax import lax
from jax.experimental import pallas as pl
from jax.experimental.pallas import tpu as pltpu

N_CHUNKS = 16
DIRECT_TAIL = 2
N_FWD = N_CHUNKS - DIRECT_TAIL


def kernel(x):
    m, n = x.shape
    half = m // 2
    ck = half // N_CHUNKS

    def body(x_ref, out_ref, ybuf, xbuf, y_send_sems, y_recv_sems,
             x_send_sems, x_recv_sems):
        my_x = lax.axis_index("x")
        my_y = lax.axis_index("y")
        my_z = lax.axis_index("z")
        y_partner = (my_x, 1 - my_y, my_z)
        x_partner = (1 - my_x, my_y, my_z)
        base = my_x * half
        obase = (1 - my_x) * half

        barrier_sem = pltpu.get_barrier_semaphore()
        for nbr in (y_partner, x_partner):
            pl.semaphore_signal(
                barrier_sem, inc=1, device_id=nbr,
                device_id_type=pl.DeviceIdType.MESH,
            )
        pl.semaphore_wait(barrier_sem, 2)

        y_rdmas = []
        for c in range(N_CHUNKS):
            rdma = pltpu.make_async_remote_copy(
                src_ref=x_ref.at[pl.ds(base + c * ck, ck), :],
                dst_ref=ybuf.at[pl.ds(c * ck, ck), :],
                send_sem=y_send_sems.at[c],
                recv_sem=y_recv_sems.at[c],
                device_id=y_partner,
                device_id_type=pl.DeviceIdType.MESH,
            )
            rdma.start()
            y_rdmas.append(rdma)
        tail_rdmas = []
        for d in range(DIRECT_TAIL):
            c = N_FWD + d
            rdma = pltpu.make_async_remote_copy(
                src_ref=x_ref.at[pl.ds(obase + c * ck, ck), :],
                dst_ref=xbuf.at[pl.ds(c * ck, ck), :],
                send_sem=y_send_sems.at[N_CHUNKS + d],
                recv_sem=x_recv_sems.at[c],
                device_id=y_partner,
                device_id_type=pl.DeviceIdType.MESH,
            )
            rdma.start()
            tail_rdmas.append(rdma)

        fwd_rdmas = []
        for c in range(N_CHUNKS):
            y_rdmas[c].wait_recv()
            if c < N_FWD:
                rdma = pltpu.make_async_remote_copy(
                    src_ref=ybuf.at[pl.ds(c * ck, ck), :],
                    dst_ref=xbuf.at[pl.ds(c * ck, ck), :],
                    send_sem=x_send_sems.at[c],
                    recv_sem=x_recv_sems.at[c],
                    device_id=x_partner,
                    device_id_type=pl.DeviceIdType.MESH,
                )
                rdma.start()
                fwd_rdmas.append(rdma)
            rows = pl.ds(base + c * ck, ck)
            out_ref[rows, :] = x_ref[rows, :] + ybuf[pl.ds(c * ck, ck), :]

        for c in range(N_CHUNKS):
            if c < N_FWD:
                fwd_rdmas[c].wait_recv()
            else:
                tail_rdmas[c - N_FWD].wait_recv()
            rows = pl.ds(obase + c * ck, ck)
            out_ref[rows, :] = x_ref[rows, :] + xbuf[pl.ds(c * ck, ck), :]

        for r in y_rdmas + tail_rdmas + fwd_rdmas:
            r.wait_send()

    return pl.pallas_call(
        body,
        out_shape=jax.ShapeDtypeStruct((m, n), x.dtype),
        in_specs=[pl.BlockSpec(memory_space=pltpu.VMEM)],
        out_specs=pl.BlockSpec(memory_space=pltpu.VMEM),
        scratch_shapes=[
            pltpu.VMEM((half, n), x.dtype),
            pltpu.VMEM((half, n), x.dtype),
            pltpu.SemaphoreType.DMA((N_CHUNKS + DIRECT_TAIL,)),
            pltpu.SemaphoreType.DMA((N_CHUNKS,)),
            pltpu.SemaphoreType.DMA((N_FWD,)),
            pltpu.SemaphoreType.DMA((N_CHUNKS,)),
        ],
        compiler_params=pltpu.CompilerParams(collective_id=0),
    )(x)
